# baseline (speedup 1.0000x reference)
import functools

import jax
import jax.numpy as jnp
from jax import lax
from jax.experimental import pallas as pl
from jax.experimental.pallas import tpu as pltpu
from jax.experimental.pallas import tpu_sc as plsc

NC = 2
NS = 16
NW = NC * NS
D = 64
C = 256
G = 128
L = 16


def _run(xt, emb, bias_f, B, F):
    n = xt.shape[0]
    n_w = n // NW
    n_chunks = n_w // C
    mesh = plsc.VectorSubcoreMesh(
        core_axis_name="c", subcore_axis_name="s", num_cores=NC, num_subcores=NS
    )

    @functools.partial(
        pl.kernel,
        out_type=jax.ShapeDtypeStruct((F, D, B), jnp.float32),
        mesh=mesh,
        scratch_types=[
            pltpu.VMEM((2, C), jnp.int32),
            pltpu.VMEM((2, C), jnp.float32),
            pltpu.VMEM((2, C, D), jnp.float32),
            pltpu.VMEM((2, D, C), jnp.float32),
            pltpu.SemaphoreType.DMA((2,)),
            pltpu.SemaphoreType.DMA((2,)),
        ],
        compiler_params=pltpu.CompilerParams(
            use_tc_tiling_on_sc=False, needs_layout_passes=False
        ),
    )
    def run(xt_hbm, emb_hbm, bias_hbm, out_hbm, idx_v, bias_v, rows_v, outT_v, gsem, osem):
        wid = lax.axis_index("s") * NC + lax.axis_index("c")
        base = wid * n_w

        def gather_copies(c, par):
            m0 = base + c * C
            copies = []
            for g in range(C // G):
                sl = pl.ds(g * G, G)
                copies.append(pltpu.make_async_copy(
                    emb_hbm.at[idx_v.at[par].at[sl]], rows_v.at[par].at[sl],
                    gsem.at[par]))
                copies.append(pltpu.make_async_copy(
                    bias_hbm.at[idx_v.at[par].at[sl]], bias_v.at[par].at[sl],
                    gsem.at[par]))
            return m0, copies

        def fire(c, par):
            m0, copies = gather_copies(c, par)
            pltpu.sync_copy(xt_hbm.at[pl.ds(m0, C)], idx_v.at[par])
            for cp in copies:
                cp.start()

        def wait_gather(c, par):
            _, copies = gather_copies(c, par)
            for cp in copies:
                cp.wait()

        def out_copy(c, par):
            m0 = base + c * C
            f = m0 // B
            b0 = m0 % B
            return pltpu.make_async_copy(
                outT_v.at[par], out_hbm.at[f].at[:, pl.ds(b0, C)], osem.at[par])

        def compute(par):
            dst = outT_v.at[par]

            def row_body(r16, _):
                bv16 = bias_v[par, pl.ds(r16 * L, L)]
                for j in range(L):
                    r = r16 * L + j
                    bv = jnp.broadcast_to(bv16[j], (L,))
                    rvec = jnp.broadcast_to(r, (L,)).astype(jnp.int32)
                    for q in range(D // L):
                        dvec = lax.iota(jnp.int32, L) + (q * L)
                        val = rows_v[par, r, pl.ds(q * L, L)] + bv
                        plsc.store_scatter(dst, [dvec, rvec], val)
                return 0

            lax.fori_loop(0, C // L, row_body, 0, unroll=False)

        fire(0, 0)

        def chunk_body(c, _):
            par = lax.rem(c, 2)

            @pl.when(c < n_chunks - 1)
            def _():
                fire(c + 1, 1 - par)

            wait_gather(c, par)

            @pl.when(c >= 2)
            def _():
                out_copy(c - 2, par).wait()

            compute(par)
            out_copy(c, par).start()
            return 0

        lax.fori_loop(0, n_chunks, chunk_body, 0, unroll=False)
        out_copy(n_chunks - 2, lax.rem(n_chunks - 2, 2)).wait()
        out_copy(n_chunks - 1, lax.rem(n_chunks - 1, 2)).wait()

    return run(xt, emb, bias_f)


def kernel(x, embedding_weight, bias_weight):
    b, f = x.shape
    n = b * f
    assert n % NW == 0 and (n // NW) % C == 0 and b % C == 0
    xt = x.T.reshape(n)
    bias_flat = bias_weight.reshape(-1)
    out3 = _run(xt, embedding_weight, bias_flat, b, f)
    return out3.transpose(2, 0, 1)


# speedup vs baseline: 1.0831x; 1.0831x over previous
import functools

import jax
import jax.numpy as jnp
from jax import lax
from jax.experimental import pallas as pl
from jax.experimental.pallas import tpu as pltpu
from jax.experimental.pallas import tpu_sc as plsc

NC = 2
NS = 16
NW = NC * NS
D = 64
C = 256
G = 128
L = 16
CP = C + 1


def _run(xt, emb2, bias_f, B, F):
    n = xt.shape[0]
    n_w = n // NW
    n_chunks = n_w // C
    mesh = plsc.VectorSubcoreMesh(
        core_axis_name="c", subcore_axis_name="s", num_cores=NC, num_subcores=NS
    )

    @functools.partial(
        pl.kernel,
        out_type=jax.ShapeDtypeStruct((F, D, B), jnp.float32),
        mesh=mesh,
        scratch_types=[
            pltpu.VMEM((2, C), jnp.int32),
            pltpu.VMEM((2, C), jnp.int32),
            pltpu.VMEM((2, C), jnp.float32),
            pltpu.VMEM((2, C, 2 * D), jnp.float32),
            pltpu.VMEM((2, D, CP), jnp.float32),
            pltpu.SemaphoreType.DMA((2,)),
            pltpu.SemaphoreType.DMA((2,)),
        ],
        compiler_params=pltpu.CompilerParams(
            use_tc_tiling_on_sc=True, needs_layout_passes=False
        ),
    )
    def run(xt_hbm, emb_hbm, bias_hbm, out_hbm,
            idx_v, pair_v, bias_v, rows_v, outT_v, gsem, osem):
        wid = lax.axis_index("s") * NC + lax.axis_index("c")
        base = wid * n_w

        def gather_copies(par):
            copies = []
            for g in range(C // G):
                sl = pl.ds(g * G, G)
                copies.append(pltpu.make_async_copy(
                    emb_hbm.at[pair_v.at[par].at[sl]], rows_v.at[par].at[sl],
                    gsem.at[par]))
                copies.append(pltpu.make_async_copy(
                    bias_hbm.at[idx_v.at[par].at[sl]], bias_v.at[par].at[sl],
                    gsem.at[par]))
            return copies

        def fire(c, par):
            m0 = base + c * C
            pltpu.sync_copy(xt_hbm.at[pl.ds(m0, C)], idx_v.at[par])
            for i in range(C // L):
                sl = pl.ds(i * L, L)
                pair_v[par, sl] = jax.lax.shift_right_logical(idx_v[par, sl], 1)
            for cp in gather_copies(par):
                cp.start()

        def out_copy(c, par):
            m0 = base + c * C
            f = m0 // B
            b0 = m0 % B
            return pltpu.make_async_copy(
                outT_v.at[par].at[:, pl.ds(0, C)],
                out_hbm.at[f].at[:, pl.ds(b0, C)],
                osem.at[par])

        def compute(par):
            dst = outT_v.at[par]

            def row_body(r16, _):
                bv16 = bias_v[par, pl.ds(r16 * L, L)]
                iv16 = idx_v[par, pl.ds(r16 * L, L)]
                for j in range(L):
                    r = r16 * L + j
                    half = (iv16[j] & 1) * D
                    bv = jnp.broadcast_to(bv16[j], (L,))
                    rvec = jnp.broadcast_to(r, (L,)).astype(jnp.int32)
                    for q in range(D // L):
                        dvec = lax.iota(jnp.int32, L) + (q * L)
                        val = rows_v[par, r, pl.ds(half + q * L, L)] + bv
                        plsc.store_scatter(dst, [dvec, rvec], val)
                return 0

            lax.fori_loop(0, C // L, row_body, 0, unroll=False)

        fire(0, 0)

        def chunk_body(c, _):
            par = lax.rem(c, 2)

            @pl.when(c < n_chunks - 1)
            def _():
                fire(c + 1, 1 - par)

            for cp in gather_copies(par):
                cp.wait()

            @pl.when(c >= 2)
            def _():
                out_copy(c - 2, par).wait()

            compute(par)
            out_copy(c, par).start()
            return 0

        lax.fori_loop(0, n_chunks, chunk_body, 0, unroll=False)
        out_copy(n_chunks - 2, lax.rem(n_chunks - 2, 2)).wait()
        out_copy(n_chunks - 1, lax.rem(n_chunks - 1, 2)).wait()

    return run(xt, emb2, bias_f)


def kernel(x, embedding_weight, bias_weight):
    b, f = x.shape
    n = b * f
    v, d = embedding_weight.shape
    assert n % NW == 0 and (n // NW) % C == 0 and b % C == 0 and d == D
    xt = x.T.reshape(n)
    emb2 = embedding_weight.reshape(v // 2, 2 * D)
    bias_flat = bias_weight.reshape(-1)
    out3 = _run(xt, emb2, bias_flat, b, f)
    return out3.transpose(2, 0, 1)


# speedup vs baseline: 1.1197x; 1.0338x over previous
import functools

import jax
import jax.numpy as jnp
from jax import lax
from jax.experimental import pallas as pl
from jax.experimental.pallas import tpu as pltpu
from jax.experimental.pallas import tpu_sc as plsc

NC = 2
NS = 16
NW = NC * NS
D = 64
C = 128
L = 16


def _run(n_w, xf, emb, bias_f, interpret=False):
    n = xf.shape[0]
    n_chunks = n_w // C
    mesh = plsc.VectorSubcoreMesh(
        core_axis_name="c", subcore_axis_name="s", num_cores=NC, num_subcores=NS
    )

    @functools.partial(
        pl.kernel,
        out_type=jax.ShapeDtypeStruct((n, D), jnp.float32),
        mesh=mesh,
        scratch_types=[
            pltpu.VMEM((2, C), jnp.int32),
            pltpu.VMEM((2, C), jnp.float32),
            pltpu.VMEM((2, C, D), jnp.float32),
            pltpu.VMEM((2, C, D), jnp.float32),
            pltpu.SemaphoreType.DMA((2,)),
            pltpu.SemaphoreType.DMA((2,)),
        ],
        interpret=interpret,
        compiler_params=pltpu.CompilerParams(use_tc_tiling_on_sc=False),
    )
    def run(xf_hbm, emb_hbm, bias_hbm, out_hbm,
            idx_v, bias_v, rows_v, obuf_v, gsem, osem):
        wid = lax.axis_index("s") * NC + lax.axis_index("c")
        base = wid * n_w

        def gather_copies(par):
            copies = []
            for i in range(C // L):
                sl = pl.ds(i * L, L)
                iv = idx_v[par, sl]
                copies.append(pltpu.make_async_copy(
                    emb_hbm.at[iv], rows_v.at[par].at[sl], gsem.at[par]))
                copies.append(pltpu.make_async_copy(
                    bias_hbm.at[iv], bias_v.at[par].at[sl], gsem.at[par]))
            return copies

        def fire(c, par):
            m0 = base + c * C
            pltpu.sync_copy(xf_hbm.at[pl.ds(m0, C)], idx_v.at[par])
            for cp in gather_copies(par):
                cp.start()

        def out_copy(c, par):
            m0 = base + c * C
            return pltpu.make_async_copy(
                obuf_v.at[par], out_hbm.at[pl.ds(m0, C)], osem.at[par])

        def compute(par):
            def row_body(r16, _):
                bv16 = bias_v[par, pl.ds(r16 * L, L)]
                for j in range(L):
                    r = r16 * L + j
                    bv = jnp.broadcast_to(bv16[j], (L,))
                    for q in range(D // L):
                        obuf_v[par, r, pl.ds(q * L, L)] = (
                            rows_v[par, r, pl.ds(q * L, L)] + bv)
                return 0

            lax.fori_loop(0, C // L, row_body, 0, unroll=False)

        fire(0, 0)

        def chunk_body(c, _):
            par = lax.rem(c, 2)

            @pl.when(c < n_chunks - 1)
            def _():
                fire(c + 1, 1 - par)

            for cp in gather_copies(par):
                cp.wait()

            @pl.when(c >= 2)
            def _():
                out_copy(c - 2, par).wait()

            compute(par)
            out_copy(c, par).start()
            return 0

        lax.fori_loop(0, n_chunks, chunk_body, 0, unroll=False)
        out_copy(n_chunks - 2, lax.rem(n_chunks - 2, 2)).wait()
        out_copy(n_chunks - 1, lax.rem(n_chunks - 1, 2)).wait()

    return run(xf, emb, bias_f)


def kernel(x, embedding_weight, bias_weight):
    b, f = x.shape
    n = b * f
    d = embedding_weight.shape[1]
    assert n % NW == 0 and (n // NW) % C == 0 and d == D
    xf = x.reshape(n)
    bias_flat = bias_weight.reshape(-1)
    out = _run(n // NW, xf, embedding_weight, bias_flat)
    return out.reshape(b, f, D)


# speedup vs baseline: 1.3227x; 1.1813x over previous
import functools

import jax
import jax.numpy as jnp
from jax import lax
from jax.experimental import pallas as pl
from jax.experimental.pallas import tpu as pltpu
from jax.experimental.pallas import tpu_sc as plsc

NC = 2
NS = 16
NW = NC * NS
D = 64
C = 128
L = 16


def _run(n_w, xf, emb, bias_f, interpret=False):
    n = xf.shape[0]
    n_chunks = n_w // C
    mesh = plsc.VectorSubcoreMesh(
        core_axis_name="c", subcore_axis_name="s", num_cores=NC, num_subcores=NS
    )

    @functools.partial(
        pl.kernel,
        out_type=jax.ShapeDtypeStruct((n, D), jnp.float32),
        mesh=mesh,
        scratch_types=[
            pltpu.VMEM((C,), jnp.int32),
            pltpu.VMEM((C,), jnp.int32),
            pltpu.VMEM((C,), jnp.float32),
            pltpu.VMEM((C,), jnp.float32),
            pltpu.VMEM((C, D), jnp.float32),
            pltpu.VMEM((C, D), jnp.float32),
            pltpu.VMEM((C, D), jnp.float32),
            pltpu.VMEM((C, D), jnp.float32),
            pltpu.SemaphoreType.DMA,
            pltpu.SemaphoreType.DMA,
            pltpu.SemaphoreType.DMA,
            pltpu.SemaphoreType.DMA,
        ],
        interpret=interpret,
        compiler_params=pltpu.CompilerParams(use_tc_tiling_on_sc=False),
    )
    def run(xf_hbm, emb_hbm, bias_hbm, out_hbm,
            idx0, idx1, bia0, bia1, row0, row1, obu0, obu1,
            gs0, gs1, os0, os1):
        idx_v = (idx0, idx1)
        bias_v = (bia0, bia1)
        rows_v = (row0, row1)
        obuf_v = (obu0, obu1)
        gsem = (gs0, gs1)
        osem = (os0, os1)

        wid = lax.axis_index("s") * NC + lax.axis_index("c")
        base = wid * n_w

        def gather_copies(par):
            return [
                pltpu.make_async_copy(
                    emb_hbm.at[idx_v[par]], rows_v[par], gsem[par]),
                pltpu.make_async_copy(
                    bias_hbm.at[idx_v[par]], bias_v[par], gsem[par]),
            ]

        def fire(c, par):
            m0 = base + c * C
            pltpu.sync_copy(xf_hbm.at[pl.ds(m0, C)], idx_v[par])
            for cp in gather_copies(par):
                cp.start()

        def out_copy(c, par):
            m0 = base + c * C
            return pltpu.make_async_copy(
                obuf_v[par], out_hbm.at[pl.ds(m0, C)], osem[par])

        def compute(par):
            def row_body(r16, _):
                bv16 = bias_v[par][pl.ds(r16 * L, L)]
                for j in range(L):
                    r = r16 * L + j
                    bv = jnp.broadcast_to(bv16[j], (L,))
                    for q in range(D // L):
                        obuf_v[par][r, pl.ds(q * L, L)] = (
                            rows_v[par][r, pl.ds(q * L, L)] + bv)
                return 0

            lax.fori_loop(0, C // L, row_body, 0, unroll=False)

        fire(0, 0)

        def chunk(c, par):
            @pl.when(c < n_chunks - 1)
            def _():
                fire(c + 1, 1 - par)

            for cp in gather_copies(par):
                cp.wait()

            @pl.when(c >= 2)
            def _():
                out_copy(c - 2, par).wait()

            compute(par)
            out_copy(c, par).start()

        def pair_body(h, _):
            chunk(2 * h, 0)
            chunk(2 * h + 1, 1)
            return 0

        lax.fori_loop(0, n_chunks // 2, pair_body, 0, unroll=False)
        out_copy(n_chunks - 2, 0).wait()
        out_copy(n_chunks - 1, 1).wait()

    return run(xf, emb, bias_f)


def kernel(x, embedding_weight, bias_weight):
    b, f = x.shape
    n = b * f
    d = embedding_weight.shape[1]
    assert n % NW == 0 and (n // NW) % (2 * C) == 0 and d == D
    xf = x.reshape(n)
    bias_flat = bias_weight.reshape(-1)
    out = _run(n // NW, xf, embedding_weight, bias_flat)
    return out.reshape(b, f, D)
